# baseline (speedup 1.0000x reference)
import jax
import jax.numpy as jnp
from jax.experimental import pallas as pl
from jax.experimental.pallas import tpu as pltpu

TEMP_INV = 10.0
EPS = 1e-8

BM = 512
BN = 512


def _info_nce_kernel(xi_ref, xj_ref, rt_ref, ct_ref, out_ref,
                     num_acc, den_acc, loss_acc):
    i = pl.program_id(0)
    j = pl.program_id(1)
    nj = pl.num_programs(1)
    ni = pl.num_programs(0)

    xi = xi_ref[...]
    xj = xj_ref[...]
    ni_norm = jnp.sqrt(jnp.sum(xi * xi, axis=1, keepdims=True))
    xi = xi / jnp.maximum(ni_norm, EPS)
    nj_norm = jnp.sqrt(jnp.sum(xj * xj, axis=1, keepdims=True))
    xj = xj / jnp.maximum(nj_norm, EPS)

    sim = jax.lax.dot_general(
        xi, xj, (((1,), (1,)), ((), ())),
        preferred_element_type=jnp.float32)
    e = jnp.exp(sim * TEMP_INV)

    bm, bn = e.shape
    row_g = jax.lax.broadcasted_iota(jnp.int32, (bm, bn), 0) + i * bm
    col_g = jax.lax.broadcasted_iota(jnp.int32, (bm, bn), 1) + j * bn
    e = jnp.where(row_g == col_g, 0.0, e)

    rt = rt_ref[0, :]
    ct = ct_ref[0, :]
    same = rt[:, None] == ct[None, :]

    num_part = jnp.sum(jnp.where(same, e, 0.0), axis=1, keepdims=True)
    den_part = jnp.sum(e, axis=1, keepdims=True)

    @pl.when(j == 0)
    def _init():
        num_acc[...] = num_part
        den_acc[...] = den_part

    @pl.when(j != 0)
    def _accum():
        num_acc[...] += num_part
        den_acc[...] += den_part

    @pl.when(j == nj - 1)
    def _finalize_rows():
        num = num_acc[...]
        den = den_acc[...]
        valid = num > 0.0
        num_safe = jnp.where(valid, num, 1.0)
        den_safe = jnp.where(den > 0.0, den, 1.0)
        losses = -jnp.log(num_safe / den_safe)
        loss_sum = jnp.sum(jnp.where(valid, losses, 0.0))
        cnt = jnp.sum(valid.astype(jnp.float32))

        @pl.when(i == 0)
        def _():
            loss_acc[0, 0] = loss_sum
            loss_acc[0, 1] = cnt

        @pl.when(i != 0)
        def _():
            loss_acc[0, 0] += loss_sum
            loss_acc[0, 1] += cnt

        @pl.when(i == ni - 1)
        def _():
            out_ref[0, 0] = loss_acc[0, 0] / jnp.maximum(loss_acc[0, 1], 1.0)


def kernel(representations, ne_tags):
    n, d = representations.shape
    tags = ne_tags.astype(jnp.int32).reshape(1, n)
    ni = n // BM
    nj = n // BN

    out = pl.pallas_call(
        _info_nce_kernel,
        grid=(ni, nj),
        in_specs=[
            pl.BlockSpec((BM, d), lambda i, j: (i, 0)),
            pl.BlockSpec((BN, d), lambda i, j: (j, 0)),
            pl.BlockSpec((1, BM), lambda i, j: (0, i)),
            pl.BlockSpec((1, BN), lambda i, j: (0, j)),
        ],
        out_specs=pl.BlockSpec(
            (1, 2), lambda i, j: (0, 0), memory_space=pltpu.SMEM),
        out_shape=jax.ShapeDtypeStruct((1, 2), jnp.float32),
        scratch_shapes=[
            pltpu.VMEM((BM, 1), jnp.float32),
            pltpu.VMEM((BM, 1), jnp.float32),
            pltpu.SMEM((1, 2), jnp.float32),
        ],
        compiler_params=pltpu.CompilerParams(
            dimension_semantics=("arbitrary", "arbitrary")),
    )(representations, representations, tags, tags)
    return out[0, 0]


# speedup vs baseline: 1.2557x; 1.2557x over previous
import jax
import jax.numpy as jnp
from jax.experimental import pallas as pl
from jax.experimental.pallas import tpu as pltpu

TEMP_INV = 10.0
EPS = 1e-8
NTAGS = 8

BM = 1024
BN = 512
BNORM = 1024


def _normalize_kernel(x_ref, out_ref):
    x = x_ref[...]
    norm = jnp.sqrt(jnp.sum(x * x, axis=1, keepdims=True))
    out_ref[...] = x / jnp.maximum(norm, EPS)


def _info_nce_kernel(xi_ref, xj_ref, rt_ref, ct_ref, out_ref,
                     r_acc, diag_acc, loss_acc):
    i = pl.program_id(0)
    j = pl.program_id(1)
    nj = pl.num_programs(1)
    ni = pl.num_programs(0)

    xi = xi_ref[...]
    xj = xj_ref[...]
    sim = jax.lax.dot_general(
        xi, xj, (((1,), (1,)), ((), ())),
        preferred_element_type=jnp.float32)
    e = jnp.exp(sim * TEMP_INV)

    ct = ct_ref[0, :]
    tag_iota = jax.lax.broadcasted_iota(jnp.int32, (e.shape[1], NTAGS), 1)
    onehot = (ct[:, None] == tag_iota).astype(jnp.float32)
    r = jax.lax.dot_general(
        e, onehot, (((1,), (0,)), ((), ())),
        preferred_element_type=jnp.float32)

    @pl.when(j == 0)
    def _init():
        r_acc[...] = r
        diag_acc[...] = jnp.zeros_like(diag_acc)

    @pl.when(j != 0)
    def _accum():
        r_acc[...] += r

    bm, bn = e.shape
    @pl.when((j * bn < (i + 1) * bm) & (i * bm < (j + 1) * bn))
    def _diag():
        row_g = jax.lax.broadcasted_iota(jnp.int32, (bm, bn), 0) + i * bm
        col_g = jax.lax.broadcasted_iota(jnp.int32, (bm, bn), 1) + j * bn
        d = jnp.sum(jnp.where(row_g == col_g, e, 0.0), axis=1, keepdims=True)
        diag_acc[...] += d

    @pl.when(j == nj - 1)
    def _finalize_rows():
        rfull = r_acc[...]
        de = diag_acc[...]
        rt = rt_ref[0, :]
        sel = (rt[:, None] ==
               jax.lax.broadcasted_iota(jnp.int32, (rfull.shape[0], NTAGS), 1))
        den = jnp.sum(rfull, axis=1, keepdims=True) - de
        num = jnp.sum(jnp.where(sel, rfull, 0.0), axis=1, keepdims=True) - de
        valid = num > 0.0
        num_safe = jnp.where(valid, num, 1.0)
        den_safe = jnp.where(den > 0.0, den, 1.0)
        losses = -jnp.log(num_safe / den_safe)
        loss_sum = jnp.sum(jnp.where(valid, losses, 0.0))
        cnt = jnp.sum(valid.astype(jnp.float32))

        @pl.when(i == 0)
        def _():
            loss_acc[0, 0] = loss_sum
            loss_acc[0, 1] = cnt

        @pl.when(i != 0)
        def _():
            loss_acc[0, 0] += loss_sum
            loss_acc[0, 1] += cnt

        @pl.when(i == ni - 1)
        def _():
            out_ref[0, 0] = loss_acc[0, 0] / jnp.maximum(loss_acc[0, 1], 1.0)


def kernel(representations, ne_tags):
    n, d = representations.shape
    tags = ne_tags.astype(jnp.int32).reshape(1, n)

    xn = pl.pallas_call(
        _normalize_kernel,
        grid=(n // BNORM,),
        in_specs=[pl.BlockSpec((BNORM, d), lambda i: (i, 0))],
        out_specs=pl.BlockSpec((BNORM, d), lambda i: (i, 0)),
        out_shape=jax.ShapeDtypeStruct((n, d), jnp.float32),
    )(representations)

    ni = n // BM
    nj = n // BN
    out = pl.pallas_call(
        _info_nce_kernel,
        grid=(ni, nj),
        in_specs=[
            pl.BlockSpec((BM, d), lambda i, j: (i, 0)),
            pl.BlockSpec((BN, d), lambda i, j: (j, 0)),
            pl.BlockSpec((1, BM), lambda i, j: (0, i)),
            pl.BlockSpec((1, BN), lambda i, j: (0, j)),
        ],
        out_specs=pl.BlockSpec(
            (1, 2), lambda i, j: (0, 0), memory_space=pltpu.SMEM),
        out_shape=jax.ShapeDtypeStruct((1, 2), jnp.float32),
        scratch_shapes=[
            pltpu.VMEM((BM, NTAGS), jnp.float32),
            pltpu.VMEM((BM, 1), jnp.float32),
            pltpu.SMEM((1, 2), jnp.float32),
        ],
        compiler_params=pltpu.CompilerParams(
            dimension_semantics=("arbitrary", "arbitrary")),
    )(xn, xn, tags, tags)
    return out[0, 0]


# speedup vs baseline: 1.2775x; 1.0174x over previous
import jax
import jax.numpy as jnp
from jax.experimental import pallas as pl
from jax.experimental.pallas import tpu as pltpu

TEMP_INV = 10.0
EPS = 1e-8
NTAGS = 8

BM = 1024
BN = 512
BNORM = 1024


def _normalize_kernel(x_ref, out_ref):
    x = x_ref[...]
    norm = jnp.sqrt(jnp.sum(x * x, axis=1, keepdims=True))
    out_ref[...] = (x / jnp.maximum(norm, EPS)).astype(jnp.bfloat16)


def _info_nce_kernel(xi_ref, xj_ref, rt_ref, ct_ref, out_ref,
                     r_acc, diag_acc, loss_acc):
    i = pl.program_id(0)
    j = pl.program_id(1)
    nj = pl.num_programs(1)
    ni = pl.num_programs(0)

    xi = xi_ref[...]
    xj = xj_ref[...]
    sim = jax.lax.dot_general(
        xi, xj, (((1,), (1,)), ((), ())),
        preferred_element_type=jnp.float32)
    e_bf = jnp.exp(sim * TEMP_INV).astype(jnp.bfloat16)

    ct = ct_ref[0, :]
    tag_iota = jax.lax.broadcasted_iota(jnp.int32, (e_bf.shape[1], NTAGS), 1)
    onehot = (ct[:, None] == tag_iota).astype(jnp.bfloat16)
    r = jax.lax.dot_general(
        e_bf, onehot, (((1,), (0,)), ((), ())),
        preferred_element_type=jnp.float32)

    @pl.when(j == 0)
    def _init():
        r_acc[...] = r
        diag_acc[...] = jnp.zeros_like(diag_acc)

    @pl.when(j != 0)
    def _accum():
        r_acc[...] += r

    bm, bn = e_bf.shape
    @pl.when((j * bn < (i + 1) * bm) & (i * bm < (j + 1) * bn))
    def _diag():
        row_g = jax.lax.broadcasted_iota(jnp.int32, (bm, bn), 0) + i * bm
        col_g = jax.lax.broadcasted_iota(jnp.int32, (bm, bn), 1) + j * bn
        d = jnp.sum(jnp.where(row_g == col_g, e_bf.astype(jnp.float32), 0.0),
                    axis=1, keepdims=True)
        diag_acc[...] += d

    @pl.when(j == nj - 1)
    def _finalize_rows():
        rfull = r_acc[...]
        de = diag_acc[...]
        rt = rt_ref[0, :]
        sel = (rt[:, None] ==
               jax.lax.broadcasted_iota(jnp.int32, (rfull.shape[0], NTAGS), 1))
        den = jnp.sum(rfull, axis=1, keepdims=True) - de
        num = jnp.sum(jnp.where(sel, rfull, 0.0), axis=1, keepdims=True) - de
        valid = num > 0.0
        num_safe = jnp.where(valid, num, 1.0)
        den_safe = jnp.where(den > 0.0, den, 1.0)
        losses = -jnp.log(num_safe / den_safe)
        loss_sum = jnp.sum(jnp.where(valid, losses, 0.0))
        cnt = jnp.sum(valid.astype(jnp.float32))

        @pl.when(i == 0)
        def _():
            loss_acc[0, 0] = loss_sum
            loss_acc[0, 1] = cnt

        @pl.when(i != 0)
        def _():
            loss_acc[0, 0] += loss_sum
            loss_acc[0, 1] += cnt

        @pl.when(i == ni - 1)
        def _():
            out_ref[0, 0] = loss_acc[0, 0] / jnp.maximum(loss_acc[0, 1], 1.0)


def kernel(representations, ne_tags):
    n, d = representations.shape
    tags = ne_tags.astype(jnp.int32).reshape(1, n)

    xn = pl.pallas_call(
        _normalize_kernel,
        grid=(n // BNORM,),
        in_specs=[pl.BlockSpec((BNORM, d), lambda i: (i, 0))],
        out_specs=pl.BlockSpec((BNORM, d), lambda i: (i, 0)),
        out_shape=jax.ShapeDtypeStruct((n, d), jnp.bfloat16),
    )(representations)

    ni = n // BM
    nj = n // BN
    out = pl.pallas_call(
        _info_nce_kernel,
        grid=(ni, nj),
        in_specs=[
            pl.BlockSpec((BM, d), lambda i, j: (i, 0)),
            pl.BlockSpec((BN, d), lambda i, j: (j, 0)),
            pl.BlockSpec((1, BM), lambda i, j: (0, i)),
            pl.BlockSpec((1, BN), lambda i, j: (0, j)),
        ],
        out_specs=pl.BlockSpec(
            (1, 2), lambda i, j: (0, 0), memory_space=pltpu.SMEM),
        out_shape=jax.ShapeDtypeStruct((1, 2), jnp.float32),
        scratch_shapes=[
            pltpu.VMEM((BM, NTAGS), jnp.float32),
            pltpu.VMEM((BM, 1), jnp.float32),
            pltpu.SMEM((1, 2), jnp.float32),
        ],
        compiler_params=pltpu.CompilerParams(
            dimension_semantics=("arbitrary", "arbitrary")),
    )(xn, xn, tags, tags)
    return out[0, 0]


# speedup vs baseline: 1.4508x; 1.1357x over previous
import jax
import jax.numpy as jnp
from jax.experimental import pallas as pl
from jax.experimental.pallas import tpu as pltpu

TEMP_INV = 10.0
EPS = 1e-8
NTAGS = 8

BM = 1024
BN = 1024
BNORM = 1024


SQRT_TINV = 3.1622776601683795


def _normalize_kernel(x_ref, out_ref):
    x = x_ref[...]
    norm = jnp.sqrt(jnp.sum(x * x, axis=1, keepdims=True))
    scale = SQRT_TINV / jnp.maximum(norm, EPS)
    out_ref[...] = (x * scale).astype(jnp.bfloat16)


def _info_nce_kernel(xi_ref, xj_ref, rt_ref, ct_ref, out_ref,
                     r_acc, diag_acc, loss_acc):
    i = pl.program_id(0)
    j = pl.program_id(1)
    nj = pl.num_programs(1)
    ni = pl.num_programs(0)

    xi = xi_ref[...]
    xj = xj_ref[...]
    sim = jax.lax.dot_general(
        xi, xj, (((1,), (1,)), ((), ())),
        preferred_element_type=jnp.float32)
    e_bf = jnp.exp(sim).astype(jnp.bfloat16)

    ct = ct_ref[0, :]
    tag_iota = jax.lax.broadcasted_iota(jnp.int32, (e_bf.shape[1], NTAGS), 1)
    onehot = (ct[:, None] == tag_iota).astype(jnp.bfloat16)
    r = jax.lax.dot_general(
        e_bf, onehot, (((1,), (0,)), ((), ())),
        preferred_element_type=jnp.float32)

    @pl.when(j == 0)
    def _init():
        r_acc[...] = r
        diag_acc[...] = jnp.zeros_like(diag_acc)

    @pl.when(j != 0)
    def _accum():
        r_acc[...] += r

    bm, bn = e_bf.shape
    @pl.when((j * bn < (i + 1) * bm) & (i * bm < (j + 1) * bn))
    def _diag():
        row_g = jax.lax.broadcasted_iota(jnp.int32, (bm, bn), 0) + i * bm
        col_g = jax.lax.broadcasted_iota(jnp.int32, (bm, bn), 1) + j * bn
        d = jnp.sum(jnp.where(row_g == col_g, e_bf.astype(jnp.float32), 0.0),
                    axis=1, keepdims=True)
        diag_acc[...] += d

    @pl.when(j == nj - 1)
    def _finalize_rows():
        rfull = r_acc[...]
        de = diag_acc[...]
        rt = rt_ref[0, :]
        sel = (rt[:, None] ==
               jax.lax.broadcasted_iota(jnp.int32, (rfull.shape[0], NTAGS), 1))
        den = jnp.sum(rfull, axis=1, keepdims=True) - de
        num = jnp.sum(jnp.where(sel, rfull, 0.0), axis=1, keepdims=True) - de
        valid = num > 0.0
        num_safe = jnp.where(valid, num, 1.0)
        den_safe = jnp.where(den > 0.0, den, 1.0)
        losses = -jnp.log(num_safe / den_safe)
        loss_sum = jnp.sum(jnp.where(valid, losses, 0.0))
        cnt = jnp.sum(valid.astype(jnp.float32))

        @pl.when(i == 0)
        def _():
            loss_acc[0, 0] = loss_sum
            loss_acc[0, 1] = cnt

        @pl.when(i != 0)
        def _():
            loss_acc[0, 0] += loss_sum
            loss_acc[0, 1] += cnt

        @pl.when(i == ni - 1)
        def _():
            out_ref[0, 0] = loss_acc[0, 0] / jnp.maximum(loss_acc[0, 1], 1.0)


def kernel(representations, ne_tags):
    n, d = representations.shape
    tags = ne_tags.astype(jnp.int32).reshape(1, n)

    xn = pl.pallas_call(
        _normalize_kernel,
        grid=(n // BNORM,),
        in_specs=[pl.BlockSpec((BNORM, d), lambda i: (i, 0))],
        out_specs=pl.BlockSpec((BNORM, d), lambda i: (i, 0)),
        out_shape=jax.ShapeDtypeStruct((n, d), jnp.bfloat16),
    )(representations)

    ni = n // BM
    nj = n // BN
    out = pl.pallas_call(
        _info_nce_kernel,
        grid=(ni, nj),
        in_specs=[
            pl.BlockSpec((BM, d), lambda i, j: (i, 0)),
            pl.BlockSpec((BN, d), lambda i, j: (j, 0)),
            pl.BlockSpec((1, BM), lambda i, j: (0, i)),
            pl.BlockSpec((1, BN), lambda i, j: (0, j)),
        ],
        out_specs=pl.BlockSpec(
            (1, 2), lambda i, j: (0, 0), memory_space=pltpu.SMEM),
        out_shape=jax.ShapeDtypeStruct((1, 2), jnp.float32),
        scratch_shapes=[
            pltpu.VMEM((BM, NTAGS), jnp.float32),
            pltpu.VMEM((BM, 1), jnp.float32),
            pltpu.SMEM((1, 2), jnp.float32),
        ],
        compiler_params=pltpu.CompilerParams(
            dimension_semantics=("arbitrary", "arbitrary")),
    )(xn, xn, tags, tags)
    return out[0, 0]
